# baseline (speedup 1.0000x reference)
import functools

import jax
import jax.numpy as jnp
from jax import lax
from jax.experimental import pallas as pl
from jax.experimental.pallas import tpu as pltpu
from jax.experimental.pallas import tpu_sc as plsc


def _gather_call(idx_flat, E, n_workers, rows_per_worker, chunk):
    Bf, = idx_flat.shape
    V, D = E.shape
    n_chunks = rows_per_worker // chunk

    mesh = plsc.VectorSubcoreMesh(core_axis_name="c", subcore_axis_name="s")

    @functools.partial(
        pl.kernel,
        mesh=mesh,
        out_type=jax.ShapeDtypeStruct((Bf, D), jnp.float32),
        scratch_types=[
            pltpu.VMEM((rows_per_worker,), jnp.int32),
            pltpu.VMEM((chunk, D), jnp.float32),
            pltpu.SemaphoreType.DMA,
        ],
        compiler_params=pltpu.CompilerParams(use_tc_tiling_on_sc=False),
    )
    def gather_kernel(idx_hbm, table_hbm, out_hbm, idx_v, rows_v, sem):
        nc = lax.axis_size("c")
        wid = lax.axis_index("s") * nc + lax.axis_index("c")
        base = wid * rows_per_worker
        pltpu.sync_copy(idx_hbm.at[pl.ds(base, rows_per_worker)], idx_v)

        def body(j, carry):
            pltpu.async_copy(
                table_hbm.at[idx_v.at[pl.ds(j * chunk, chunk)]], rows_v, sem
            ).wait()
            pltpu.sync_copy(rows_v, out_hbm.at[pl.ds(base + j * chunk, chunk)])
            return carry

        lax.fori_loop(0, n_chunks, body, 0)

    return gather_kernel(idx_flat, E)


def kernel(x, E):
    B, H = x.shape
    V, D = E.shape
    Bf = B * H
    info = plsc.get_sparse_core_info()
    n_workers = info.num_cores * info.num_subcores
    rows_per_worker = Bf // n_workers
    out = _gather_call(x.reshape(Bf).astype(jnp.int32), E,
                       n_workers, rows_per_worker, chunk=128)
    return out.reshape(B, H, D)


# speedup vs baseline: 1.1142x; 1.1142x over previous
import functools

import jax
import jax.numpy as jnp
from jax import lax
from jax.experimental import pallas as pl
from jax.experimental.pallas import tpu as pltpu
from jax.experimental.pallas import tpu_sc as plsc


def _gather_call(idx_flat, E, n_workers, rows_per_worker, chunk, nbuf):
    Bf, = idx_flat.shape
    V, D = E.shape
    n_chunks = rows_per_worker // chunk
    n_groups = n_chunks // nbuf
    assert n_chunks % nbuf == 0

    mesh = plsc.VectorSubcoreMesh(core_axis_name="c", subcore_axis_name="s")

    @functools.partial(
        pl.kernel,
        mesh=mesh,
        out_type=jax.ShapeDtypeStruct((Bf, D), jnp.float32),
        scratch_types=(
            [pltpu.VMEM((rows_per_worker,), jnp.int32)]
            + [pltpu.VMEM((chunk, D), jnp.float32) for _ in range(nbuf)]
            + [pltpu.SemaphoreType.DMA for _ in range(2 * nbuf)]
        ),
        compiler_params=pltpu.CompilerParams(use_tc_tiling_on_sc=False),
    )
    def gather_kernel(idx_hbm, table_hbm, out_hbm, idx_v, *bufs_and_sems):
        bufs = bufs_and_sems[:nbuf]
        gsems = bufs_and_sems[nbuf:2 * nbuf]
        osems = bufs_and_sems[2 * nbuf:]
        nc = lax.axis_size("c")
        wid = lax.axis_index("s") * nc + lax.axis_index("c")
        base = wid * rows_per_worker
        pltpu.sync_copy(idx_hbm.at[pl.ds(base, rows_per_worker)], idx_v)

        def start_gather(b, j):
            pltpu.async_copy(
                table_hbm.at[idx_v.at[pl.ds(j * chunk, chunk)]], bufs[b],
                gsems[b])

        def wait_gather(b):
            pltpu.make_async_copy(
                table_hbm.at[pl.ds(0, chunk)], bufs[b], gsems[b]).wait()

        def start_out(b, j):
            pltpu.async_copy(
                bufs[b], out_hbm.at[pl.ds(base + j * chunk, chunk)], osems[b])

        def wait_out(b):
            pltpu.make_async_copy(
                bufs[b], out_hbm.at[pl.ds(base, chunk)], osems[b]).wait()

        for b in range(nbuf):
            start_gather(b, b)

        def group(g, carry):
            j_prev = (g - 1) * nbuf
            j_next = g * nbuf
            for b in range(nbuf):
                wait_gather(b)
                start_out(b, j_prev + b)
            for b in range(nbuf):
                wait_out(b)
                start_gather(b, j_next + b)
            return carry

        lax.fori_loop(1, n_groups, group, 0)

        j_last = (n_groups - 1) * nbuf
        for b in range(nbuf):
            wait_gather(b)
            start_out(b, j_last + b)
        for b in range(nbuf):
            wait_out(b)

    return gather_kernel(idx_flat, E)


def kernel(x, E):
    B, H = x.shape
    V, D = E.shape
    Bf = B * H
    info = plsc.get_sparse_core_info()
    n_workers = info.num_cores * info.num_subcores
    rows_per_worker = Bf // n_workers
    out = _gather_call(x.reshape(Bf).astype(jnp.int32), E,
                       n_workers, rows_per_worker, chunk=128, nbuf=8)
    return out.reshape(B, H, D)


# speedup vs baseline: 1.4813x; 1.3295x over previous
import functools

import jax
import jax.numpy as jnp
from jax import lax
from jax.experimental import pallas as pl
from jax.experimental.pallas import tpu as pltpu
from jax.experimental.pallas import tpu_sc as plsc


def _gather_call(idx_flat, E, n_workers, rows_per_worker, chunk, nbuf):
    Bf, = idx_flat.shape
    V, D = E.shape
    n_chunks = rows_per_worker // chunk
    n_groups = n_chunks // nbuf
    assert n_chunks % nbuf == 0

    mesh = plsc.VectorSubcoreMesh(core_axis_name="c", subcore_axis_name="s")

    @functools.partial(
        pl.kernel,
        mesh=mesh,
        out_type=jax.ShapeDtypeStruct((Bf, 2 * D), jnp.float32),
        scratch_types=(
            [pltpu.VMEM((rows_per_worker,), jnp.int32)]
            + [pltpu.VMEM((chunk, D), jnp.float32) for _ in range(nbuf)]
            + [pltpu.SemaphoreType.DMA for _ in range(2 * nbuf)]
        ),
        compiler_params=pltpu.CompilerParams(use_tc_tiling_on_sc=False),
    )
    def gather_kernel(idx_hbm, table_hbm, out_hbm, idx_v, *bufs_and_sems):
        bufs = bufs_and_sems[:nbuf]
        gsems = bufs_and_sems[nbuf:2 * nbuf]
        osems = bufs_and_sems[2 * nbuf:]
        nc = lax.axis_size("c")
        wid = lax.axis_index("s") * nc + lax.axis_index("c")
        base = wid * rows_per_worker
        pltpu.sync_copy(idx_hbm.at[pl.ds(base, rows_per_worker)], idx_v)

        def start_gather(b, j):
            pltpu.async_copy(
                table_hbm.at[idx_v.at[pl.ds(j * chunk, chunk)]], bufs[b],
                gsems[b])

        def wait_gather(b):
            pltpu.make_async_copy(
                table_hbm.at[pl.ds(0, chunk)], bufs[b], gsems[b]).wait()

        def start_out(b, j):
            pltpu.async_copy(
                bufs[b],
                out_hbm.at[pl.ds(base + j * chunk, chunk), pl.ds(0, D)],
                osems[b])

        def wait_out(b):
            pltpu.make_async_copy(
                bufs[b], out_hbm.at[pl.ds(base, chunk), pl.ds(0, D)],
                osems[b]).wait()

        for b in range(nbuf):
            start_gather(b, b)

        def group(g, carry):
            j_prev = (g - 1) * nbuf
            j_next = g * nbuf
            for b in range(nbuf):
                wait_gather(b)
                start_out(b, j_prev + b)
            for b in range(nbuf):
                wait_out(b)
                start_gather(b, j_next + b)
            return carry

        lax.fori_loop(1, n_groups, group, 0)

        j_last = (n_groups - 1) * nbuf
        for b in range(nbuf):
            wait_gather(b)
            start_out(b, j_last + b)
        for b in range(nbuf):
            wait_out(b)

    return gather_kernel(idx_flat, E)


def kernel(x, E):
    B, H = x.shape
    V, D = E.shape
    Bf = B * H
    info = plsc.get_sparse_core_info()
    n_workers = info.num_cores * info.num_subcores
    rows_per_worker = Bf // n_workers
    out = _gather_call(x.reshape(Bf).astype(jnp.int32), E,
                       n_workers, rows_per_worker, chunk=128, nbuf=5)
    return out.reshape(B, H, 2 * D)[:, :, :D]
